# baseline (speedup 1.0000x reference)
import functools

import jax
import jax.numpy as jnp
from jax import lax
from jax.experimental import pallas as pl
from jax.experimental.pallas import tpu as pltpu
from jax.experimental.pallas import tpu_sc as plsc

B, N, M = 16, 2048, 1024
NWORKERS = 32
MW = M // 2
CH = 128
NCHUNKS = N // CH
L = 16
MGROUPS = MW // L


def _sc_kernel_body(x_hbm, out_hbm, buf, minv, mini):
    c = lax.axis_index("c")
    s = lax.axis_index("s")
    wid = s * 2 + c
    b = wid // 2
    m0 = (wid % 2) * MW

    def init_group(j, _):
        base = j * L
        minv[pl.ds(base, L)] = jnp.full((L,), jnp.inf, jnp.float32)
        mini[pl.ds(base, L)] = jnp.zeros((L,), jnp.int32)
        return _

    lax.fori_loop(0, MGROUPS, init_group, None)

    def chunk_step(chunk, _):
        n0 = chunk * CH
        pltpu.sync_copy(x_hbm.at[b, pl.ds(n0, CH), pl.ds(m0, MW)], buf)

        def col_group(j, _):
            base = j * L

            def row_step(n, carry):
                cmin, cidx = carry
                v = buf[n, pl.ds(base, L)]
                nvec = jnp.full((L,), n0 + n, jnp.int32)
                p = v < cmin
                return jnp.where(p, v, cmin), jnp.where(p, nvec, cidx)

            cmin, cidx = lax.fori_loop(
                0, CH, row_step, (minv[pl.ds(base, L)], mini[pl.ds(base, L)]))
            minv[pl.ds(base, L)] = cmin
            mini[pl.ds(base, L)] = cidx
            return _

        lax.fori_loop(0, MGROUPS, col_group, None)
        return _

    lax.fori_loop(0, NCHUNKS, chunk_step, None)
    pltpu.sync_copy(mini, out_hbm.at[b, pl.ds(m0, MW)])


@functools.partial(
    pl.kernel,
    mesh=plsc.VectorSubcoreMesh(core_axis_name="c", subcore_axis_name="s"),
    out_type=jax.ShapeDtypeStruct((B, M), jnp.int32),
    scratch_types=[
        pltpu.VMEM((CH, MW), jnp.float32),
        pltpu.VMEM((MW,), jnp.float32),
        pltpu.VMEM((MW,), jnp.int32),
    ],
)
def _sc_argmin(x_hbm, out_hbm, buf, minv, mini):
    _sc_kernel_body(x_hbm, out_hbm, buf, minv, mini)


def kernel(x):
    return _sc_argmin(x)


# speedup vs baseline: 2.7274x; 2.7274x over previous
import functools

import jax
import jax.numpy as jnp
from jax import lax
from jax.experimental import pallas as pl
from jax.experimental.pallas import tpu as pltpu
from jax.experimental.pallas import tpu_sc as plsc

B, N, M = 16, 2048, 1024
MW = M // 2
CH = 64
NCHUNKS = N // CH
L = 16
MGROUPS = MW // L
U = 8


def _sc_kernel_body(x_hbm, out_hbm, bufs, minv, mini, sem0, sem1):
    c = lax.axis_index("c")
    s = lax.axis_index("s")
    wid = s * 2 + c
    b = wid // 2
    m0 = (wid % 2) * MW
    sems = (sem0, sem1)

    def init_group(j, _):
        base = j * L
        minv[pl.ds(base, L)] = jnp.full((L,), jnp.inf, jnp.float32)
        mini[pl.ds(base, L)] = jnp.zeros((L,), jnp.int32)
        return _

    lax.fori_loop(0, MGROUPS, init_group, None)

    def start_copy(chunk, slot):
        return pltpu.async_copy(
            x_hbm.at[b, pl.ds(chunk * CH, CH), pl.ds(m0, MW)],
            bufs.at[slot], sems[slot])

    def compute(slot, n0):
        def col_group(j, _):
            base = j * L

            def row_blk(i, carry):
                cmin, cidx = carry
                nb = i * U
                for k in range(U):
                    v = bufs[slot, nb + k, pl.ds(base, L)]
                    nvec = jnp.full((L,), n0 + nb + k, jnp.int32)
                    p = v < cmin
                    cmin = jnp.where(p, v, cmin)
                    cidx = jnp.where(p, nvec, cidx)
                return cmin, cidx

            cmin, cidx = lax.fori_loop(
                0, CH // U, row_blk,
                (minv[pl.ds(base, L)], mini[pl.ds(base, L)]))
            minv[pl.ds(base, L)] = cmin
            mini[pl.ds(base, L)] = cidx
            return _

        lax.fori_loop(0, MGROUPS, col_group, None)

    copies = {0: start_copy(0, 0), 1: start_copy(1, 1)}
    for chunk in range(NCHUNKS):
        slot = chunk % 2
        copies[chunk].wait()
        compute(slot, chunk * CH)
        if chunk + 2 < NCHUNKS:
            copies[chunk + 2] = start_copy(chunk + 2, slot)

    pltpu.sync_copy(mini, out_hbm.at[b, pl.ds(m0, MW)])


@functools.partial(
    pl.kernel,
    mesh=plsc.VectorSubcoreMesh(core_axis_name="c", subcore_axis_name="s"),
    out_type=jax.ShapeDtypeStruct((B, M), jnp.int32),
    scratch_types=[
        pltpu.VMEM((2, CH, MW), jnp.float32),
        pltpu.VMEM((MW,), jnp.float32),
        pltpu.VMEM((MW,), jnp.int32),
        pltpu.SemaphoreType.DMA,
        pltpu.SemaphoreType.DMA,
    ],
)
def _sc_argmin(x_hbm, out_hbm, bufs, minv, mini, sem0, sem1):
    _sc_kernel_body(x_hbm, out_hbm, bufs, minv, mini, sem0, sem1)


def kernel(x):
    return _sc_argmin(x)


# speedup vs baseline: 3.5480x; 1.3009x over previous
import functools

import jax
import jax.numpy as jnp
from jax import lax
from jax.experimental import pallas as pl
from jax.experimental.pallas import tpu as pltpu
from jax.experimental.pallas import tpu_sc as plsc

B, N, M = 16, 2048, 1024
MW = M // 2
CH = 64
NCHUNKS = N // CH
L = 16
MGROUPS = MW // L
U = 8
A = 4


def _sc_kernel_body(x_hbm, out_hbm, bufs, minacc, idxacc, mini, sem0, sem1):
    c = lax.axis_index("c")
    s = lax.axis_index("s")
    wid = s * 2 + c
    b = wid // 2
    m0 = (wid % 2) * MW
    sems = (sem0, sem1)

    def init_group(j, _):
        base = j * L
        for k in range(A):
            minacc[k, pl.ds(base, L)] = jnp.full((L,), jnp.inf, jnp.float32)
            idxacc[k, pl.ds(base, L)] = jnp.zeros((L,), jnp.int32)
        return _

    lax.fori_loop(0, MGROUPS, init_group, None)

    def start_copy(chunk, slot):
        return pltpu.async_copy(
            x_hbm.at[b, pl.ds(chunk * CH, CH), pl.ds(m0, MW)],
            bufs.at[slot], sems[slot])

    def compute(slot, n0):
        def col_group(j, _):
            base = j * L

            def row_blk(i, carry):
                mins, idxs = carry
                mins, idxs = list(mins), list(idxs)
                nb = i * U
                for h in range(U // A):
                    tvec = jnp.full((L,), (n0 + nb) // A + h, jnp.int32)
                    for a in range(A):
                        v = bufs[slot, nb + h * A + a, pl.ds(base, L)]
                        p = v < mins[a]
                        mins[a] = jnp.where(p, v, mins[a])
                        idxs[a] = jnp.where(p, tvec, idxs[a])
                return tuple(mins), tuple(idxs)

            carry0 = (tuple(minacc[k, pl.ds(base, L)] for k in range(A)),
                      tuple(idxacc[k, pl.ds(base, L)] for k in range(A)))
            mins, idxs = lax.fori_loop(0, CH // U, row_blk, carry0)
            for k in range(A):
                minacc[k, pl.ds(base, L)] = mins[k]
                idxacc[k, pl.ds(base, L)] = idxs[k]
            return _

        lax.fori_loop(0, MGROUPS, col_group, None)

    copies = {0: start_copy(0, 0), 1: start_copy(1, 1)}
    for chunk in range(NCHUNKS):
        slot = chunk % 2
        copies[chunk].wait()
        compute(slot, chunk * CH)
        if chunk + 2 < NCHUNKS:
            copies[chunk + 2] = start_copy(chunk + 2, slot)

    def merge_group(j, _):
        base = j * L
        bv = minacc[0, pl.ds(base, L)]
        bi = idxacc[0, pl.ds(base, L)] * A
        for k in range(1, A):
            v = minacc[k, pl.ds(base, L)]
            i = idxacc[k, pl.ds(base, L)] * A + k
            p = (v < bv) | ((v == bv) & (i < bi))
            bv = jnp.where(p, v, bv)
            bi = jnp.where(p, i, bi)
        mini[pl.ds(base, L)] = bi
        return _

    lax.fori_loop(0, MGROUPS, merge_group, None)
    pltpu.sync_copy(mini, out_hbm.at[b, pl.ds(m0, MW)])


@functools.partial(
    pl.kernel,
    mesh=plsc.VectorSubcoreMesh(core_axis_name="c", subcore_axis_name="s"),
    out_type=jax.ShapeDtypeStruct((B, M), jnp.int32),
    scratch_types=[
        pltpu.VMEM((2, CH, MW), jnp.float32),
        pltpu.VMEM((A, MW), jnp.float32),
        pltpu.VMEM((A, MW), jnp.int32),
        pltpu.VMEM((MW,), jnp.int32),
        pltpu.SemaphoreType.DMA,
        pltpu.SemaphoreType.DMA,
    ],
)
def _sc_argmin(x_hbm, out_hbm, bufs, minacc, idxacc, mini, sem0, sem1):
    _sc_kernel_body(x_hbm, out_hbm, bufs, minacc, idxacc, mini, sem0, sem1)


def kernel(x):
    return _sc_argmin(x)
